# baseline (speedup 1.0000x reference)
import functools

import jax
import jax.numpy as jnp
from jax.experimental import pallas as pl
from jax.experimental.pallas import tpu as pltpu
from jax.experimental.pallas import tpu_sc as plsc

T, D, F, E, K = 2048, 1024, 512, 64, 2
BT = 64
NB = T * K // BT + E
NBT = NB * BT
NW = 32
SEG = NBT // NW
CH = 16
TOK_W = T // NW


def _router_body(x_ref, gw_ref, logits_ref, sel_ref, pos_ref, wv_ref,
                 eid_ref, nbu_ref, loss_ref, xcp_ref):
    x = x_ref[...]
    gw = gw_ref[...]
    xcp_ref[...] = x
    logits = jax.lax.dot_general(x, gw, (((1,), (1,)), ((), ())),
                                 preferred_element_type=jnp.float32)
    logits_ref[...] = logits

    m = jnp.max(logits, axis=1, keepdims=True)
    eu = jnp.exp(logits - m)
    s = jnp.sum(eu, axis=1, keepdims=True)
    probs = eu / s

    eiota = jax.lax.broadcasted_iota(jnp.int32, (T, E), 1)
    m1 = jnp.max(probs, axis=1, keepdims=True)
    i1 = jnp.min(jnp.where(probs == m1, eiota, E), axis=1, keepdims=True)
    pm = jnp.where(eiota == i1, -jnp.inf, probs)
    m2 = jnp.max(pm, axis=1, keepdims=True)
    i2 = jnp.min(jnp.where(pm == m2, eiota, E), axis=1, keepdims=True)
    wsum = m1 + m2
    w1n = m1 / wsum
    w2n = m2 / wsum

    sel_ref[...] = jnp.concatenate([i1, i2], axis=1)
    wv_ref[...] = jnp.concatenate([w1n, w2n], axis=1)

    sel1 = eiota == i1
    sel2 = eiota == i2
    imp = jnp.sum(jnp.where(sel1, w1n, 0.0) + jnp.where(sel2, w2n, 0.0),
                  axis=0, keepdims=True)
    mean = jnp.sum(imp) * (1.0 / E)
    var = jnp.sum((imp - mean) ** 2) * (1.0 / (E - 1))
    loss_ref[...] = jnp.reshape(jnp.sqrt(var), (1, 1))

    M = jnp.where(sel1, 1.0, 0.0) + jnp.where(sel2, 1.0, 0.0)
    cum = M
    k = 1
    while k < T:
        cum = cum + jnp.concatenate(
            [jnp.zeros((k, E), jnp.float32), cum[:T - k]], axis=0)
        k *= 2
    cum = cum - M
    c_row = jnp.sum(M, axis=0, keepdims=True)
    pc_row = jnp.floor((c_row + (BT - 1.0)) * (1.0 / BT))
    bb = pc_row
    k = 1
    while k < E:
        bb = bb + jnp.concatenate(
            [jnp.zeros((1, k), jnp.float32), bb[:, :E - k]], axis=1)
        k *= 2
    bb = bb - pc_row
    row_start = bb * BT
    addr = row_start + cum
    p1 = jnp.sum(jnp.where(sel1, addr, 0.0), axis=1, keepdims=True)
    p2 = jnp.sum(jnp.where(sel2, addr, 0.0), axis=1, keepdims=True)
    pos_ref[...] = jnp.concatenate([p1, p2], axis=1).astype(jnp.int32)

    nbu = jnp.sum(pc_row)
    nbu_ref[...] = jnp.reshape(nbu, (1, 1)).astype(jnp.int32)

    bc = jax.lax.broadcasted_iota(jnp.int32, (1, NB), 1).astype(jnp.float32)
    bc = jnp.minimum(bc, nbu - 1.0)
    ones_col = jnp.zeros((T, 1), jnp.float32) + 1.0
    c_col = jax.lax.dot_general(M, ones_col, (((0,), (0,)), ((), ())),
                                preferred_element_type=jnp.float32)
    pc_col = jnp.floor((c_col + (BT - 1.0)) * (1.0 / BT))
    bbc = pc_col
    k = 1
    while k < E:
        bbc = bbc + jnp.concatenate(
            [jnp.zeros((k, 1), jnp.float32), bbc[:E - k]], axis=0)
        k *= 2
    bbc = bbc - pc_col
    cmp = jnp.where(bbc <= bc, 1.0, 0.0)
    eid = jnp.sum(cmp, axis=0, keepdims=True) - 1.0
    eid_ref[...] = eid.astype(jnp.int32)


_router_call = pl.pallas_call(
    _router_body,
    out_shape=(
        jax.ShapeDtypeStruct((T, E), jnp.float32),
        jax.ShapeDtypeStruct((T, K), jnp.int32),
        jax.ShapeDtypeStruct((T, K), jnp.int32),
        jax.ShapeDtypeStruct((T, K), jnp.float32),
        jax.ShapeDtypeStruct((1, NB), jnp.int32),
        jax.ShapeDtypeStruct((1, 1), jnp.int32),
        jax.ShapeDtypeStruct((1, 1), jnp.float32),
        jax.ShapeDtypeStruct((T, D), jnp.float32),
    ),
)


_NBUF = 4
_LA = 2


def _dispatch_body(x_hbm, pos_hbm, wv_hbm, xg_hbm, ws_hbm,
                   pos_v, wv_v, gseg_v, wseg_v,
                   buf0, buf1, buf2, buf3, semg, semo):
    cid = jax.lax.axis_index("c")
    sid = jax.lax.axis_index("s")
    wid = sid * 2 + cid
    base = wid * SEG
    bufs = (buf0, buf1, buf2, buf3)

    pltpu.sync_copy(pos_hbm, pos_v)
    pltpu.sync_copy(wv_hbm, wv_v)

    for j in range(SEG // 16):
        gseg_v[pl.ds(j * 16, 16)] = ((jax.lax.iota(jnp.int32, 16) +
                                      (base + j * 16)) & (T - 1))
        wseg_v[pl.ds(j * 16, 16)] = jnp.zeros((16,), jnp.float32)

    def scat(j, carry):
        pv = pos_v[pl.ds(j * 16, 16)]
        tok = jax.lax.shift_right_logical(
            jax.lax.iota(jnp.int32, 16) + j * 16, 1)
        rel = pv - base
        m = (rel >= 0) & (rel < SEG)
        relc = jnp.minimum(jnp.maximum(rel, 0), SEG - 1)
        plsc.store_scatter(gseg_v, [relc], tok, mask=m)
        plsc.store_scatter(wseg_v, [relc], wv_v[pl.ds(j * 16, 16)], mask=m)
        return carry
    jax.lax.fori_loop(0, T * K // 16, scat, 0)

    nch = SEG // CH

    def gstart(k):
        return pltpu.async_copy(x_hbm.at[gseg_v.at[pl.ds(k * CH, CH)]],
                                bufs[k % _NBUF], semg)
    gs = {k: gstart(k) for k in range(_LA)}
    os_ = {}
    for k in range(nch):
        gs[k].wait()
        os_[k] = pltpu.async_copy(bufs[k % _NBUF],
                                  xg_hbm.at[pl.ds(base + k * CH, CH)], semo)
        nk = k + _LA
        if nk < nch:
            if nk - _NBUF >= 0:
                os_[nk - _NBUF].wait()
            gs[nk] = gstart(nk)
    for k in range(max(0, nch - _NBUF), nch):
        os_[k].wait()

    pltpu.sync_copy(wseg_v, ws_hbm.at[pl.ds(base, SEG)])


@functools.cache
def _dispatch_call():
  return pl.kernel(
    _dispatch_body,
    out_type=(
        jax.ShapeDtypeStruct((NBT, D), jnp.float32),
        jax.ShapeDtypeStruct((NBT,), jnp.float32),
    ),
    mesh=plsc.VectorSubcoreMesh(core_axis_name="c", subcore_axis_name="s"),
    compiler_params=pltpu.CompilerParams(needs_layout_passes=False),
    scratch_types=[
        pltpu.VMEM((T * K,), jnp.int32),
        pltpu.VMEM((T * K,), jnp.float32),
        pltpu.VMEM((SEG,), jnp.int32),
        pltpu.VMEM((SEG,), jnp.float32),
        pltpu.VMEM((CH, D), jnp.float32),
        pltpu.VMEM((CH, D), jnp.float32),
        pltpu.VMEM((CH, D), jnp.float32),
        pltpu.VMEM((CH, D), jnp.float32),
        pltpu.SemaphoreType.DMA,
        pltpu.SemaphoreType.DMA,
    ],
  )


def _expert_body(eid_ref, nbu_ref, xg_ref, w1_ref, w3_ref, w2_ref, ws_ref,
                 yg_ref):
    b = pl.program_id(0)

    @pl.when(b < nbu_ref[0])
    def _():
        x = xg_ref[...]
        a = jax.lax.dot_general(x, w1_ref[0], (((1,), (1,)), ((), ())),
                                preferred_element_type=jnp.float32)
        c = jax.lax.dot_general(x, w3_ref[0], (((1,), (1,)), ((), ())),
                                preferred_element_type=jnp.float32)
        h = (a * (1.0 / (1.0 + jnp.exp(-a)))) * c
        y = jax.lax.dot_general(h, w2_ref[0], (((1,), (1,)), ((), ())),
                                preferred_element_type=jnp.float32)
        yg_ref[...] = y * ws_ref[0]


def _clamped(b, eid, nbu):
    return (jnp.minimum(b, nbu[0] - 1), 0)


_expert_call = pl.pallas_call(
    _expert_body,
    grid_spec=pltpu.PrefetchScalarGridSpec(
        num_scalar_prefetch=2,
        grid=(NB,),
        in_specs=[
            pl.BlockSpec((BT, D), _clamped),
            pl.BlockSpec((1, F, D), lambda b, eid, nbu: (eid[b], 0, 0)),
            pl.BlockSpec((1, F, D), lambda b, eid, nbu: (eid[b], 0, 0)),
            pl.BlockSpec((1, D, F), lambda b, eid, nbu: (eid[b], 0, 0)),
            pl.BlockSpec((1, BT, 1),
                         lambda b, eid, nbu: (jnp.minimum(b, nbu[0] - 1), 0, 0)),
        ],
        out_specs=pl.BlockSpec((BT, D), _clamped),
    ),
    out_shape=jax.ShapeDtypeStruct((NBT, D), jnp.float32),
)


def _combine_body(yg_hbm, pos_hbm, out_hbm,
                  ib, ra, rb, oa, ob, semg, semo):
    cid = jax.lax.axis_index("c")
    sid = jax.lax.axis_index("s")
    wid = sid * 2 + cid
    tb = wid * TOK_W
    rs = (ra, rb)
    ovs = (oa, ob)
    tpc = CH // 2
    nch = TOK_W // tpc

    pltpu.sync_copy(pos_hbm.at[pl.ds(tb * K, TOK_W * K)], ib)

    def gstart(k):
        return pltpu.async_copy(yg_hbm.at[ib.at[pl.ds(k * CH, CH)]],
                                rs[k % 2], semg)
    gs = {k: gstart(k) for k in range(min(2, nch))}
    os_ = {}
    for k in range(nch):
        gs[k].wait()
        if k >= 2:
            os_[k - 2].wait()
        rv, ov = rs[k % 2], ovs[k % 2]

        def add_row(i, carry):
            for j in range(D // 16):
                ov[i, pl.ds(j * 16, 16)] = (rv[2 * i, pl.ds(j * 16, 16)] +
                                            rv[2 * i + 1, pl.ds(j * 16, 16)])
            return carry
        jax.lax.fori_loop(0, tpc, add_row, 0)
        os_[k] = pltpu.async_copy(ov, out_hbm.at[pl.ds(tb + k * tpc, tpc)],
                                  semo)
        if k + 2 < nch:
            gs[k + 2] = gstart(k + 2)
    for k in range(max(0, nch - 2), nch):
        os_[k].wait()


@functools.cache
def _combine_call():
  return pl.kernel(
    _combine_body,
    out_type=jax.ShapeDtypeStruct((T, D), jnp.float32),
    mesh=plsc.VectorSubcoreMesh(core_axis_name="c", subcore_axis_name="s"),
    scratch_types=[
        pltpu.VMEM((TOK_W * K,), jnp.int32),
        pltpu.VMEM((CH, D), jnp.float32),
        pltpu.VMEM((CH, D), jnp.float32),
        pltpu.VMEM((CH // 2, D), jnp.float32),
        pltpu.VMEM((CH // 2, D), jnp.float32),
        pltpu.SemaphoreType.DMA,
        pltpu.SemaphoreType.DMA,
    ],
  )


@jax.jit
def kernel(hidden_states, gate_w, w1, w3, w2):
    logits, sel, pos, wv, eid, nbu, loss, xcp = _router_call(hidden_states,
                                                             gate_w)
    posf = pos.reshape(T * K)
    wvf = wv.reshape(T * K)
    xg, ws = _dispatch_call()(xcp, posf, wvf)
    yg = _expert_call(eid.reshape(NB), nbu.reshape(1), xg, w1, w3, w2,
                      ws.reshape(NB, BT, 1))
    final = _combine_call()(yg, posf)
    return final, logits, sel, loss[0, 0]


# speedup vs baseline: 1.0363x; 1.0363x over previous
import functools

import jax
import jax.numpy as jnp
from jax.experimental import pallas as pl
from jax.experimental.pallas import tpu as pltpu
from jax.experimental.pallas import tpu_sc as plsc

T, D, F, E, K = 2048, 1024, 512, 64, 2
BT = 64
NB = T * K // BT + E
NBT = NB * BT
NW = 32
SEG = NBT // NW
CH = 16
TOK_W = T // NW


def _router_body(x_ref, gw_ref, logits_ref, sel_ref, pos_ref, wv_ref,
                 eid_ref, nbu_ref, loss_ref, xcp_ref):
    x = x_ref[...]
    gw = gw_ref[...]
    xcp_ref[...] = x
    logits = jax.lax.dot_general(x, gw, (((1,), (1,)), ((), ())),
                                 preferred_element_type=jnp.float32)
    logits_ref[...] = logits

    m = jnp.max(logits, axis=1, keepdims=True)
    eu = jnp.exp(logits - m)
    s = jnp.sum(eu, axis=1, keepdims=True)
    probs = eu / s

    eiota = jax.lax.broadcasted_iota(jnp.int32, (T, E), 1)
    m1 = jnp.max(probs, axis=1, keepdims=True)
    i1 = jnp.min(jnp.where(probs == m1, eiota, E), axis=1, keepdims=True)
    pm = jnp.where(eiota == i1, -jnp.inf, probs)
    m2 = jnp.max(pm, axis=1, keepdims=True)
    i2 = jnp.min(jnp.where(pm == m2, eiota, E), axis=1, keepdims=True)
    wsum = m1 + m2
    w1n = m1 / wsum
    w2n = m2 / wsum

    sel_ref[...] = jnp.concatenate([i1, i2], axis=1)
    wv_ref[...] = jnp.concatenate([w1n, w2n], axis=1)

    sel1 = eiota == i1
    sel2 = eiota == i2
    imp = jnp.sum(jnp.where(sel1, w1n, 0.0) + jnp.where(sel2, w2n, 0.0),
                  axis=0, keepdims=True)
    mean = jnp.sum(imp) * (1.0 / E)
    var = jnp.sum((imp - mean) ** 2) * (1.0 / (E - 1))
    loss_ref[...] = jnp.reshape(jnp.sqrt(var), (1, 1))

    M = jnp.where(sel1, 1.0, 0.0) + jnp.where(sel2, 1.0, 0.0)
    cum = M
    k = 1
    while k < T:
        cum = cum + jnp.concatenate(
            [jnp.zeros((k, E), jnp.float32), cum[:T - k]], axis=0)
        k *= 2
    cum = cum - M
    c_row = jnp.sum(M, axis=0, keepdims=True)
    pc_row = jnp.floor((c_row + (BT - 1.0)) * (1.0 / BT))
    bb = pc_row
    k = 1
    while k < E:
        bb = bb + jnp.concatenate(
            [jnp.zeros((1, k), jnp.float32), bb[:, :E - k]], axis=1)
        k *= 2
    bb = bb - pc_row
    row_start = bb * BT
    addr = row_start + cum
    p1 = jnp.sum(jnp.where(sel1, addr, 0.0), axis=1, keepdims=True)
    p2 = jnp.sum(jnp.where(sel2, addr, 0.0), axis=1, keepdims=True)
    pos_ref[...] = jnp.concatenate([p1, p2], axis=1).astype(jnp.int32)

    nbu = jnp.sum(pc_row)
    nbu_ref[...] = jnp.reshape(nbu, (1, 1)).astype(jnp.int32)

    bc = jax.lax.broadcasted_iota(jnp.int32, (1, NB), 1).astype(jnp.float32)
    bc = jnp.minimum(bc, nbu - 1.0)
    ones_col = jnp.zeros((T, 1), jnp.float32) + 1.0
    c_col = jax.lax.dot_general(M, ones_col, (((0,), (0,)), ((), ())),
                                preferred_element_type=jnp.float32)
    pc_col = jnp.floor((c_col + (BT - 1.0)) * (1.0 / BT))
    bbc = pc_col
    k = 1
    while k < E:
        bbc = bbc + jnp.concatenate(
            [jnp.zeros((k, 1), jnp.float32), bbc[:E - k]], axis=0)
        k *= 2
    bbc = bbc - pc_col
    cmp = jnp.where(bbc <= bc, 1.0, 0.0)
    eid = jnp.sum(cmp, axis=0, keepdims=True) - 1.0
    eid_ref[...] = eid.astype(jnp.int32)


_router_call = pl.pallas_call(
    _router_body,
    out_shape=(
        jax.ShapeDtypeStruct((T, E), jnp.float32),
        jax.ShapeDtypeStruct((T, K), jnp.int32),
        jax.ShapeDtypeStruct((T, K), jnp.int32),
        jax.ShapeDtypeStruct((T, K), jnp.float32),
        jax.ShapeDtypeStruct((1, NB), jnp.int32),
        jax.ShapeDtypeStruct((1, 1), jnp.int32),
        jax.ShapeDtypeStruct((1, 1), jnp.float32),
        jax.ShapeDtypeStruct((T, D), jnp.float32),
    ),
)


_NBUF = 4
_LA = 2


def _dispatch_body(x_hbm, p1_hbm, p2_hbm, wa_hbm, wb_hbm, xg_hbm, ws_hbm,
                   p1_v, p2_v, wa_v, wb_v, gseg_v, wseg_v,
                   buf0, buf1, buf2, buf3, semg, semo):
    cid = jax.lax.axis_index("c")
    sid = jax.lax.axis_index("s")
    wid = sid * 2 + cid
    base = wid * SEG
    bufs = (buf0, buf1, buf2, buf3)

    pltpu.sync_copy(p1_hbm, p1_v)
    pltpu.sync_copy(p2_hbm, p2_v)
    pltpu.sync_copy(wa_hbm, wa_v)
    pltpu.sync_copy(wb_hbm, wb_v)

    for j in range(SEG // 16):
        gseg_v[pl.ds(j * 16, 16)] = ((jax.lax.iota(jnp.int32, 16) +
                                      (base + j * 16)) & (T - 1))
        wseg_v[pl.ds(j * 16, 16)] = jnp.zeros((16,), jnp.float32)

    def scat(j, carry):
        tok = jax.lax.iota(jnp.int32, 16) + j * 16
        for pv_ref, wv_ref in ((p1_v, wa_v), (p2_v, wb_v)):
            pv = pv_ref[pl.ds(j * 16, 16)]
            rel = pv - base
            m = (rel >= 0) & (rel < SEG)
            relc = jnp.minimum(jnp.maximum(rel, 0), SEG - 1)
            plsc.store_scatter(gseg_v, [relc], tok, mask=m)
            plsc.store_scatter(wseg_v, [relc], wv_ref[pl.ds(j * 16, 16)],
                               mask=m)
        return carry
    jax.lax.fori_loop(0, T // 16, scat, 0)

    nch = SEG // CH

    def gstart(k):
        return pltpu.async_copy(x_hbm.at[gseg_v.at[pl.ds(k * CH, CH)]],
                                bufs[k % _NBUF], semg)
    gs = {k: gstart(k) for k in range(_LA)}
    os_ = {}
    for k in range(nch):
        gs[k].wait()
        os_[k] = pltpu.async_copy(bufs[k % _NBUF],
                                  xg_hbm.at[pl.ds(base + k * CH, CH)], semo)
        nk = k + _LA
        if nk < nch:
            if nk - _NBUF >= 0:
                os_[nk - _NBUF].wait()
            gs[nk] = gstart(nk)
    for k in range(max(0, nch - _NBUF), nch):
        os_[k].wait()

    pltpu.sync_copy(wseg_v, ws_hbm.at[pl.ds(base, SEG)])


@functools.cache
def _dispatch_call():
  return pl.kernel(
    _dispatch_body,
    out_type=(
        jax.ShapeDtypeStruct((NBT, D), jnp.float32),
        jax.ShapeDtypeStruct((NBT,), jnp.float32),
    ),
    mesh=plsc.VectorSubcoreMesh(core_axis_name="c", subcore_axis_name="s"),
    compiler_params=pltpu.CompilerParams(needs_layout_passes=False),
    scratch_types=[
        pltpu.VMEM((T,), jnp.int32),
        pltpu.VMEM((T,), jnp.int32),
        pltpu.VMEM((T,), jnp.float32),
        pltpu.VMEM((T,), jnp.float32),
        pltpu.VMEM((SEG,), jnp.int32),
        pltpu.VMEM((SEG,), jnp.float32),
        pltpu.VMEM((CH, D), jnp.float32),
        pltpu.VMEM((CH, D), jnp.float32),
        pltpu.VMEM((CH, D), jnp.float32),
        pltpu.VMEM((CH, D), jnp.float32),
        pltpu.SemaphoreType.DMA,
        pltpu.SemaphoreType.DMA,
    ],
  )


def _expert_body(eid_ref, nbu_ref, xg_ref, w1_ref, w3_ref, w2_ref, ws_ref,
                 yg_ref):
    b = pl.program_id(0)

    @pl.when(b < nbu_ref[0])
    def _():
        x = xg_ref[...]
        a = jax.lax.dot_general(x, w1_ref[0], (((1,), (1,)), ((), ())),
                                preferred_element_type=jnp.float32)
        c = jax.lax.dot_general(x, w3_ref[0], (((1,), (1,)), ((), ())),
                                preferred_element_type=jnp.float32)
        h = (a * (1.0 / (1.0 + jnp.exp(-a)))) * c
        y = jax.lax.dot_general(h, w2_ref[0], (((1,), (1,)), ((), ())),
                                preferred_element_type=jnp.float32)
        yg_ref[...] = y * ws_ref[0]


def _clamped(b, eid, nbu):
    return (jnp.minimum(b, nbu[0] - 1), 0)


_expert_call = pl.pallas_call(
    _expert_body,
    grid_spec=pltpu.PrefetchScalarGridSpec(
        num_scalar_prefetch=2,
        grid=(NB,),
        in_specs=[
            pl.BlockSpec((BT, D), _clamped),
            pl.BlockSpec((1, F, D), lambda b, eid, nbu: (eid[b], 0, 0)),
            pl.BlockSpec((1, F, D), lambda b, eid, nbu: (eid[b], 0, 0)),
            pl.BlockSpec((1, D, F), lambda b, eid, nbu: (eid[b], 0, 0)),
            pl.BlockSpec((1, BT, 1),
                         lambda b, eid, nbu: (jnp.minimum(b, nbu[0] - 1), 0, 0)),
        ],
        out_specs=pl.BlockSpec((BT, D), _clamped),
    ),
    out_shape=jax.ShapeDtypeStruct((NBT, D), jnp.float32),
)


def _combine_body(yg_hbm, p1_hbm, p2_hbm, out_hbm,
                  i1_b, i2_b, r1a, r1b, r2a, r2b, oa, ob, semg, semo):
    cid = jax.lax.axis_index("c")
    sid = jax.lax.axis_index("s")
    wid = sid * 2 + cid
    tb = wid * TOK_W
    r1s = (r1a, r1b)
    r2s = (r2a, r2b)
    ovs = (oa, ob)
    nch = TOK_W // CH

    pltpu.sync_copy(p1_hbm.at[pl.ds(tb, TOK_W)], i1_b)
    pltpu.sync_copy(p2_hbm.at[pl.ds(tb, TOK_W)], i2_b)

    def gstart(k):
        return (pltpu.async_copy(yg_hbm.at[i1_b.at[pl.ds(k * CH, CH)]],
                                 r1s[k % 2], semg),
                pltpu.async_copy(yg_hbm.at[i2_b.at[pl.ds(k * CH, CH)]],
                                 r2s[k % 2], semg))
    gs = {k: gstart(k) for k in range(min(2, nch))}
    os_ = {}
    for k in range(nch):
        ga, gb = gs[k]
        ga.wait()
        gb.wait()
        if k >= 2:
            os_[k - 2].wait()
        r1v, r2v, ov = r1s[k % 2], r2s[k % 2], ovs[k % 2]

        def add_row(i, carry):
            for j in range(D // 16):
                ov[i, pl.ds(j * 16, 16)] = (r1v[i, pl.ds(j * 16, 16)] +
                                            r2v[i, pl.ds(j * 16, 16)])
            return carry
        jax.lax.fori_loop(0, CH, add_row, 0)
        os_[k] = pltpu.async_copy(ov, out_hbm.at[pl.ds(tb + k * CH, CH)],
                                  semo)
        if k + 2 < nch:
            gs[k + 2] = gstart(k + 2)
    for k in range(max(0, nch - 2), nch):
        os_[k].wait()


@functools.cache
def _combine_call():
  return pl.kernel(
    _combine_body,
    out_type=jax.ShapeDtypeStruct((T, D), jnp.float32),
    mesh=plsc.VectorSubcoreMesh(core_axis_name="c", subcore_axis_name="s"),
    scratch_types=[
        pltpu.VMEM((TOK_W,), jnp.int32),
        pltpu.VMEM((TOK_W,), jnp.int32),
        pltpu.VMEM((CH, D), jnp.float32),
        pltpu.VMEM((CH, D), jnp.float32),
        pltpu.VMEM((CH, D), jnp.float32),
        pltpu.VMEM((CH, D), jnp.float32),
        pltpu.VMEM((CH, D), jnp.float32),
        pltpu.VMEM((CH, D), jnp.float32),
        pltpu.SemaphoreType.DMA,
        pltpu.SemaphoreType.DMA,
    ],
  )


@jax.jit
def kernel(hidden_states, gate_w, w1, w3, w2):
    logits, sel, pos, wv, eid, nbu, loss, xcp = _router_call(hidden_states,
                                                             gate_w)
    p1 = pos[:, 0]
    p2 = pos[:, 1]
    xg, ws = _dispatch_call()(xcp, p1, p2, wv[:, 0], wv[:, 1])
    yg = _expert_call(eid.reshape(NB), nbu.reshape(1), xg, w1, w3, w2,
                      ws.reshape(NB, BT, 1))
    final = _combine_call()(yg, p1, p2)
    return final, logits, sel, loss[0, 0]
